# Initial kernel scaffold; baseline (speedup 1.0000x reference)
import jax
import jax.numpy as jnp
from jax.experimental import pallas as pl


def kernel(x, edge_index, Wc0, Wc1, Wd0, Wd1, alphas):
    raise NotImplementedError("write your pallas kernel here")


# speedup vs baseline: 6.2384x; 6.2384x over previous
import functools

import jax
import jax.numpy as jnp
from jax import lax
from jax.experimental import pallas as pl
from jax.experimental.pallas import tpu as pltpu
from jax.experimental.pallas import tpu_sc as plsc

N = 10000
E = 320000
D = 128
NC, NS = 2, 16
NT = NC * NS
CHUNK = 128
NPAD = 10240
RPT = NPAD // NS
DUMMY = N
KF = 80
KH = 40
DW = 16

_mesh = functools.partial(
    plsc.VectorSubcoreMesh, core_axis_name="c", subcore_axis_name="s",
    num_cores=NC, num_subcores=NS)


def _propagate_body(z_hbm, src_hbm, dst_hbm, out_hbm, src_v, dst_v, buf_v,
                    acc_sh, sem, K):
    c = lax.axis_index("c")
    s = lax.axis_index("s")
    wid = c * NS + s
    base = s * RPT

    pltpu.sync_copy(src_hbm.at[pl.ds(wid * K, K)], src_v)
    pltpu.sync_copy(dst_hbm.at[pl.ds(wid * K, K)], dst_v)

    zero16 = jnp.zeros((16,), jnp.float32)

    def _zrow(i, carry):
        for l in range(D // 16):
            buf_v[i, pl.ds(l * 16, 16)] = zero16
        return carry

    lax.fori_loop(0, CHUNK, _zrow, 0)
    for b in range(RPT // CHUNK):
        pltpu.sync_copy(buf_v, acc_sh.at[pl.ds(base + b * CHUNK, CHUNK)])
    plsc.subcore_barrier()

    def _edge(j, carry):
        pltpu.async_copy(z_hbm.at[src_v.at[j]], buf_v, sem).wait()
        pltpu.sync_copy(buf_v, acc_sh.at[dst_v.at[j]], add=True)
        return carry

    lax.fori_loop(0, K, _edge, 0)
    plsc.subcore_barrier()

    pltpu.sync_copy(acc_sh.at[pl.ds(base, RPT)],
                    out_hbm.at[c, pl.ds(base, RPT)])


def _make_propagate(K):
    body = functools.partial(_propagate_body, K=K)
    return pl.kernel(
        body,
        out_type=jax.ShapeDtypeStruct((NC, NPAD, D), jnp.float32),
        mesh=_mesh(),
        scratch_types=[
            pltpu.VMEM((K, CHUNK), jnp.int32),
            pltpu.VMEM((K, CHUNK), jnp.int32),
            pltpu.VMEM((CHUNK, D), jnp.float32),
            pltpu.VMEM_SHARED((NPAD, D), jnp.float32),
            pltpu.SemaphoreType.DMA,
        ],
    )


def _degree_body(dste_hbm, dsto_hbm, oute_hbm, outo_hbm, dste_v, dsto_v,
                 buf_v, acce_sh, acco_sh):
    c = lax.axis_index("c")
    s = lax.axis_index("s")
    wid = c * NS + s
    base = s * RPT
    K = KH

    pltpu.sync_copy(dste_hbm.at[pl.ds(wid * K, K)], dste_v)
    pltpu.sync_copy(dsto_hbm.at[pl.ds(wid * K, K)], dsto_v)

    zero16 = jnp.zeros((16,), jnp.float32)

    def _zrow(i, carry):
        buf_v[i, :] = zero16
        return carry

    lax.fori_loop(0, CHUNK, _zrow, 0)
    for b in range(RPT // CHUNK):
        pltpu.sync_copy(buf_v, acce_sh.at[pl.ds(base + b * CHUNK, CHUNK)])
        pltpu.sync_copy(buf_v, acco_sh.at[pl.ds(base + b * CHUNK, CHUNK)])
    plsc.subcore_barrier()

    one16 = jnp.ones((16,), jnp.float32)

    def _orow(i, carry):
        buf_v[i, :] = one16
        return carry

    lax.fori_loop(0, CHUNK, _orow, 0)

    def _edge(j, carry):
        pltpu.sync_copy(buf_v, acce_sh.at[dste_v.at[j]], add=True)
        pltpu.sync_copy(buf_v, acco_sh.at[dsto_v.at[j]], add=True)
        return carry

    lax.fori_loop(0, K, _edge, 0)
    plsc.subcore_barrier()

    pltpu.sync_copy(acce_sh.at[pl.ds(base, RPT)],
                    oute_hbm.at[c, pl.ds(base, RPT)])
    pltpu.sync_copy(acco_sh.at[pl.ds(base, RPT)],
                    outo_hbm.at[c, pl.ds(base, RPT)])


def _make_degree():
    return pl.kernel(
        _degree_body,
        out_type=(jax.ShapeDtypeStruct((NC, NPAD, DW), jnp.float32),
                  jax.ShapeDtypeStruct((NC, NPAD, DW), jnp.float32)),
        mesh=_mesh(),
        scratch_types=[
            pltpu.VMEM((KH, CHUNK), jnp.int32),
            pltpu.VMEM((KH, CHUNK), jnp.int32),
            pltpu.VMEM((CHUNK, DW), jnp.float32),
            pltpu.VMEM_SHARED((NPAD, DW), jnp.float32),
            pltpu.VMEM_SHARED((NPAD, DW), jnp.float32),
        ],
    )


_TCBLK = 2560


def _dinv_z_body(cnte_ref, cnto_ref, x_ref, dinvf_ref, dinvh_ref, z_ref):
    ce = cnte_ref[0, :, 0:1] + cnte_ref[1, :, 0:1]
    co = cnto_ref[0, :, 0:1] + cnto_ref[1, :, 0:1]
    dinvf = lax.rsqrt(ce + co + 1.0)
    dinvh = lax.rsqrt(ce + 1.0)
    dinvf_ref[...] = dinvf
    dinvh_ref[...] = dinvh
    z_ref[...] = dinvf * x_ref[...]


def _dinv_z(cnte, cnto, x_pad):
    return pl.pallas_call(
        _dinv_z_body,
        grid=(NPAD // _TCBLK,),
        in_specs=[
            pl.BlockSpec((NC, _TCBLK, DW), lambda i: (0, i, 0)),
            pl.BlockSpec((NC, _TCBLK, DW), lambda i: (0, i, 0)),
            pl.BlockSpec((_TCBLK, D), lambda i: (i, 0)),
        ],
        out_specs=[
            pl.BlockSpec((_TCBLK, 1), lambda i: (i, 0)),
            pl.BlockSpec((_TCBLK, 1), lambda i: (i, 0)),
            pl.BlockSpec((_TCBLK, D), lambda i: (i, 0)),
        ],
        out_shape=[
            jax.ShapeDtypeStruct((NPAD, 1), jnp.float32),
            jax.ShapeDtypeStruct((NPAD, 1), jnp.float32),
            jax.ShapeDtypeStruct((NPAD, D), jnp.float32),
        ],
    )(cnte, cnto, x_pad)


def _conv_body(acc_ref, z_ref, dinv_ref, dinvn_ref, w_ref, h_ref, zn_ref):
    u = acc_ref[0] + acc_ref[1] + z_ref[...]
    t = dinv_ref[...] * u
    h = jax.nn.relu(lax.dot_general(
        t, w_ref[...], (((1,), (0,)), ((), ())),
        precision=lax.Precision.HIGHEST, preferred_element_type=jnp.float32))
    h_ref[...] = h
    zn_ref[...] = dinvn_ref[...] * h


def _conv(acc, z, dinv, dinv_next, w):
    return pl.pallas_call(
        _conv_body,
        grid=(NPAD // _TCBLK,),
        in_specs=[
            pl.BlockSpec((NC, _TCBLK, D), lambda i: (0, i, 0)),
            pl.BlockSpec((_TCBLK, D), lambda i: (i, 0)),
            pl.BlockSpec((_TCBLK, 1), lambda i: (i, 0)),
            pl.BlockSpec((_TCBLK, 1), lambda i: (i, 0)),
            pl.BlockSpec((D, D), lambda i: (0, 0)),
        ],
        out_specs=[
            pl.BlockSpec((_TCBLK, D), lambda i: (i, 0)),
            pl.BlockSpec((_TCBLK, D), lambda i: (i, 0)),
        ],
        out_shape=[
            jax.ShapeDtypeStruct((NPAD, D), jnp.float32),
            jax.ShapeDtypeStruct((NPAD, D), jnp.float32),
        ],
    )(acc, z, dinv, dinv_next, w)


def _conv_blend_body(acc_ref, z_ref, h_ref, dinv_ref, dinvn_ref, w_ref,
                     a_ref, hn_ref, zn_ref):
    u = acc_ref[0] + acc_ref[1] + z_ref[...]
    t = dinv_ref[...] * u
    nh = jax.nn.relu(lax.dot_general(
        t, w_ref[...], (((1,), (0,)), ((), ())),
        precision=lax.Precision.HIGHEST, preferred_element_type=jnp.float32))
    a = a_ref[0, 0]
    hn = a * nh + (1.0 - a) * h_ref[...]
    hn_ref[...] = hn
    zn_ref[...] = dinvn_ref[...] * hn


def _conv_blend(acc, z, h, dinv, dinv_next, w, a):
    return pl.pallas_call(
        _conv_blend_body,
        grid=(NPAD // _TCBLK,),
        in_specs=[
            pl.BlockSpec((NC, _TCBLK, D), lambda i: (0, i, 0)),
            pl.BlockSpec((_TCBLK, D), lambda i: (i, 0)),
            pl.BlockSpec((_TCBLK, D), lambda i: (i, 0)),
            pl.BlockSpec((_TCBLK, 1), lambda i: (i, 0)),
            pl.BlockSpec((_TCBLK, 1), lambda i: (i, 0)),
            pl.BlockSpec((D, D), lambda i: (0, 0)),
            pl.BlockSpec(memory_space=pltpu.SMEM),
        ],
        out_specs=[
            pl.BlockSpec((_TCBLK, D), lambda i: (i, 0)),
            pl.BlockSpec((_TCBLK, D), lambda i: (i, 0)),
        ],
        out_shape=[
            jax.ShapeDtypeStruct((NPAD, D), jnp.float32),
            jax.ShapeDtypeStruct((NPAD, D), jnp.float32),
        ],
    )(acc, z, h, dinv, dinv_next, w, a)


def _conv_final_body(acc_ref, z_ref, h_ref, skip_ref, dinv_ref, w_ref,
                     a_ref, out_ref):
    u = acc_ref[0] + acc_ref[1] + z_ref[...]
    t = dinv_ref[...] * u
    nh = jax.nn.relu(lax.dot_general(
        t, w_ref[...], (((1,), (0,)), ((), ())),
        precision=lax.Precision.HIGHEST, preferred_element_type=jnp.float32))
    a = a_ref[0, 0]
    out_ref[...] = a * nh + (1.0 - a) * h_ref[...] + skip_ref[...]


def _conv_final(acc, z, h, skip, dinv, w, a):
    return pl.pallas_call(
        _conv_final_body,
        grid=(NPAD // _TCBLK,),
        in_specs=[
            pl.BlockSpec((NC, _TCBLK, D), lambda i: (0, i, 0)),
            pl.BlockSpec((_TCBLK, D), lambda i: (i, 0)),
            pl.BlockSpec((_TCBLK, D), lambda i: (i, 0)),
            pl.BlockSpec((_TCBLK, D), lambda i: (i, 0)),
            pl.BlockSpec((_TCBLK, 1), lambda i: (i, 0)),
            pl.BlockSpec((D, D), lambda i: (0, 0)),
            pl.BlockSpec(memory_space=pltpu.SMEM),
        ],
        out_specs=pl.BlockSpec((_TCBLK, D), lambda i: (i, 0)),
        out_shape=jax.ShapeDtypeStruct((NPAD, D), jnp.float32),
    )(acc, z, h, skip, dinv, w, a)


def _pad_idx(idx, k):
    total = NT * k * CHUNK
    idx = jnp.concatenate(
        [idx, jnp.full((total - idx.shape[0],), DUMMY, jnp.int32)])
    return idx.reshape(NT * k, CHUNK)


def kernel(x, edge_index, Wc0, Wc1, Wd0, Wd1, alphas):
    src = edge_index[0].astype(jnp.int32)
    dst = edge_index[1].astype(jnp.int32)

    x_pad = jnp.zeros((NPAD, D), jnp.float32).at[:N].set(x)
    srcf = _pad_idx(src, KF)
    dstf = _pad_idx(dst, KF)
    srch = _pad_idx(src[0::2], KH)
    dsth = _pad_idx(dst[0::2], KH)
    dste = dsth
    dsto = _pad_idx(dst[1::2], KH)

    prop_full = _make_propagate(KF)
    prop_half = _make_propagate(KH)

    cnte, cnto = _make_degree()(dste, dsto)
    dinvf, dinvh, z = _dinv_z(cnte, cnto, x_pad)

    a = jax.nn.sigmoid(alphas.astype(jnp.float32))
    a0 = a[0].reshape(1, 1)
    a1 = a[1].reshape(1, 1)

    acc = prop_full(z, srcf, dstf)
    h1, z = _conv(acc, z, dinvf, dinvf, Wc0)

    acc = prop_full(z, srcf, dstf)
    h2, z = _conv(acc, z, dinvf, dinvf, Wc1)

    acc = prop_full(z, srcf, dstf)
    h3, z = _conv_blend(acc, z, h2, dinvf, dinvh, Wd0, a0)

    acc = prop_half(z, srch, dsth)
    out = _conv_final(acc, z, h3, h2, dinvh, Wd1, a1)

    return out[:N]


# speedup vs baseline: 6.2442x; 1.0009x over previous
import functools

import jax
import jax.numpy as jnp
from jax import lax
from jax.experimental import pallas as pl
from jax.experimental.pallas import tpu as pltpu
from jax.experimental.pallas import tpu_sc as plsc

N = 10000
E = 320000
D = 128
NC, NS = 2, 16
NT = NC * NS
CHUNK = 128
NPAD = 10240
RPT = NPAD // NS
DUMMY = N
KF = 80
KH = 40
DW = 16

_mesh = functools.partial(
    plsc.VectorSubcoreMesh, core_axis_name="c", subcore_axis_name="s",
    num_cores=NC, num_subcores=NS)


NBUF = 1


def _propagate_body(z_hbm, src_hbm, dst_hbm, out_hbm, src_v, dst_v, buf_v,
                    acc_sh, *sems, K):
    c = lax.axis_index("c")
    s = lax.axis_index("s")
    wid = c * NS + s
    base = s * RPT

    pltpu.sync_copy(src_hbm.at[pl.ds(wid * K, K)], src_v)
    pltpu.sync_copy(dst_hbm.at[pl.ds(wid * K, K)], dst_v)

    zero16 = jnp.zeros((16,), jnp.float32)

    def _zrow(i, carry):
        for l in range(D // 16):
            buf_v[0, i, pl.ds(l * 16, 16)] = zero16
        return carry

    lax.fori_loop(0, CHUNK, _zrow, 0)
    for b in range(RPT // CHUNK):
        pltpu.sync_copy(buf_v.at[0], acc_sh.at[pl.ds(base + b * CHUNK, CHUNK)])

    for b in range(NBUF):
        pltpu.async_copy(z_hbm.at[src_v.at[b]], buf_v.at[b], sems[b])
    plsc.subcore_barrier()

    rounds = K // NBUF

    def _round(g, carry):
        for b in range(NBUF):
            j = g * NBUF + b
            pltpu.make_async_copy(
                z_hbm.at[src_v.at[j]], buf_v.at[b], sems[b]).wait()
            pltpu.sync_copy(buf_v.at[b], acc_sh.at[dst_v.at[j]], add=True)
            pltpu.async_copy(z_hbm.at[src_v.at[j + NBUF]], buf_v.at[b],
                             sems[b])
        return carry

    lax.fori_loop(0, rounds - 1, _round, 0)
    for b in range(NBUF):
        j = (rounds - 1) * NBUF + b
        pltpu.make_async_copy(
            z_hbm.at[src_v.at[j]], buf_v.at[b], sems[b]).wait()
        pltpu.sync_copy(buf_v.at[b], acc_sh.at[dst_v.at[j]], add=True)
    plsc.subcore_barrier()

    pltpu.sync_copy(acc_sh.at[pl.ds(base, RPT)],
                    out_hbm.at[c, pl.ds(base, RPT)])


def _make_propagate(K):
    body = functools.partial(_propagate_body, K=K)
    return pl.kernel(
        body,
        out_type=jax.ShapeDtypeStruct((NC, NPAD, D), jnp.float32),
        mesh=_mesh(),
        scratch_types=[
            pltpu.VMEM((K, CHUNK), jnp.int32),
            pltpu.VMEM((K, CHUNK), jnp.int32),
            pltpu.VMEM((NBUF, CHUNK, D), jnp.float32),
            pltpu.VMEM_SHARED((NPAD, D), jnp.float32),
        ] + [pltpu.SemaphoreType.DMA] * NBUF,
    )


def _degree_body(dste_hbm, dsto_hbm, oute_hbm, outo_hbm, dste_v, dsto_v,
                 buf_v, acce_sh, acco_sh):
    c = lax.axis_index("c")
    s = lax.axis_index("s")
    wid = c * NS + s
    base = s * RPT
    K = KH

    pltpu.sync_copy(dste_hbm.at[pl.ds(wid * K, K)], dste_v)
    pltpu.sync_copy(dsto_hbm.at[pl.ds(wid * K, K)], dsto_v)

    zero16 = jnp.zeros((16,), jnp.float32)

    def _zrow(i, carry):
        buf_v[i, :] = zero16
        return carry

    lax.fori_loop(0, CHUNK, _zrow, 0)
    for b in range(RPT // CHUNK):
        pltpu.sync_copy(buf_v, acce_sh.at[pl.ds(base + b * CHUNK, CHUNK)])
        pltpu.sync_copy(buf_v, acco_sh.at[pl.ds(base + b * CHUNK, CHUNK)])
    plsc.subcore_barrier()

    one16 = jnp.ones((16,), jnp.float32)

    def _orow(i, carry):
        buf_v[i, :] = one16
        return carry

    lax.fori_loop(0, CHUNK, _orow, 0)

    def _edge(j, carry):
        pltpu.sync_copy(buf_v, acce_sh.at[dste_v.at[j]], add=True)
        pltpu.sync_copy(buf_v, acco_sh.at[dsto_v.at[j]], add=True)
        return carry

    lax.fori_loop(0, K, _edge, 0)
    plsc.subcore_barrier()

    pltpu.sync_copy(acce_sh.at[pl.ds(base, RPT)],
                    oute_hbm.at[c, pl.ds(base, RPT)])
    pltpu.sync_copy(acco_sh.at[pl.ds(base, RPT)],
                    outo_hbm.at[c, pl.ds(base, RPT)])


def _make_degree():
    return pl.kernel(
        _degree_body,
        out_type=(jax.ShapeDtypeStruct((NC, NPAD, DW), jnp.float32),
                  jax.ShapeDtypeStruct((NC, NPAD, DW), jnp.float32)),
        mesh=_mesh(),
        scratch_types=[
            pltpu.VMEM((KH, CHUNK), jnp.int32),
            pltpu.VMEM((KH, CHUNK), jnp.int32),
            pltpu.VMEM((CHUNK, DW), jnp.float32),
            pltpu.VMEM_SHARED((NPAD, DW), jnp.float32),
            pltpu.VMEM_SHARED((NPAD, DW), jnp.float32),
        ],
    )


_TCBLK = 2560


def _dinv_z_body(cnte_ref, cnto_ref, x_ref, dinvf_ref, dinvh_ref, z_ref):
    ce = cnte_ref[0, :, 0:1] + cnte_ref[1, :, 0:1]
    co = cnto_ref[0, :, 0:1] + cnto_ref[1, :, 0:1]
    dinvf = lax.rsqrt(ce + co + 1.0)
    dinvh = lax.rsqrt(ce + 1.0)
    dinvf_ref[...] = dinvf
    dinvh_ref[...] = dinvh
    z_ref[...] = dinvf * x_ref[...]


def _dinv_z(cnte, cnto, x_pad):
    return pl.pallas_call(
        _dinv_z_body,
        grid=(NPAD // _TCBLK,),
        in_specs=[
            pl.BlockSpec((NC, _TCBLK, DW), lambda i: (0, i, 0)),
            pl.BlockSpec((NC, _TCBLK, DW), lambda i: (0, i, 0)),
            pl.BlockSpec((_TCBLK, D), lambda i: (i, 0)),
        ],
        out_specs=[
            pl.BlockSpec((_TCBLK, 1), lambda i: (i, 0)),
            pl.BlockSpec((_TCBLK, 1), lambda i: (i, 0)),
            pl.BlockSpec((_TCBLK, D), lambda i: (i, 0)),
        ],
        out_shape=[
            jax.ShapeDtypeStruct((NPAD, 1), jnp.float32),
            jax.ShapeDtypeStruct((NPAD, 1), jnp.float32),
            jax.ShapeDtypeStruct((NPAD, D), jnp.float32),
        ],
    )(cnte, cnto, x_pad)


def _conv_body(acc_ref, z_ref, dinv_ref, dinvn_ref, w_ref, h_ref, zn_ref):
    u = acc_ref[0] + acc_ref[1] + z_ref[...]
    t = dinv_ref[...] * u
    h = jax.nn.relu(lax.dot_general(
        t, w_ref[...], (((1,), (0,)), ((), ())),
        precision=lax.Precision.HIGHEST, preferred_element_type=jnp.float32))
    h_ref[...] = h
    zn_ref[...] = dinvn_ref[...] * h


def _conv(acc, z, dinv, dinv_next, w):
    return pl.pallas_call(
        _conv_body,
        grid=(NPAD // _TCBLK,),
        in_specs=[
            pl.BlockSpec((NC, _TCBLK, D), lambda i: (0, i, 0)),
            pl.BlockSpec((_TCBLK, D), lambda i: (i, 0)),
            pl.BlockSpec((_TCBLK, 1), lambda i: (i, 0)),
            pl.BlockSpec((_TCBLK, 1), lambda i: (i, 0)),
            pl.BlockSpec((D, D), lambda i: (0, 0)),
        ],
        out_specs=[
            pl.BlockSpec((_TCBLK, D), lambda i: (i, 0)),
            pl.BlockSpec((_TCBLK, D), lambda i: (i, 0)),
        ],
        out_shape=[
            jax.ShapeDtypeStruct((NPAD, D), jnp.float32),
            jax.ShapeDtypeStruct((NPAD, D), jnp.float32),
        ],
    )(acc, z, dinv, dinv_next, w)


def _conv_blend_body(acc_ref, z_ref, h_ref, dinv_ref, dinvn_ref, w_ref,
                     a_ref, hn_ref, zn_ref):
    u = acc_ref[0] + acc_ref[1] + z_ref[...]
    t = dinv_ref[...] * u
    nh = jax.nn.relu(lax.dot_general(
        t, w_ref[...], (((1,), (0,)), ((), ())),
        precision=lax.Precision.HIGHEST, preferred_element_type=jnp.float32))
    a = a_ref[0, 0]
    hn = a * nh + (1.0 - a) * h_ref[...]
    hn_ref[...] = hn
    zn_ref[...] = dinvn_ref[...] * hn


def _conv_blend(acc, z, h, dinv, dinv_next, w, a):
    return pl.pallas_call(
        _conv_blend_body,
        grid=(NPAD // _TCBLK,),
        in_specs=[
            pl.BlockSpec((NC, _TCBLK, D), lambda i: (0, i, 0)),
            pl.BlockSpec((_TCBLK, D), lambda i: (i, 0)),
            pl.BlockSpec((_TCBLK, D), lambda i: (i, 0)),
            pl.BlockSpec((_TCBLK, 1), lambda i: (i, 0)),
            pl.BlockSpec((_TCBLK, 1), lambda i: (i, 0)),
            pl.BlockSpec((D, D), lambda i: (0, 0)),
            pl.BlockSpec(memory_space=pltpu.SMEM),
        ],
        out_specs=[
            pl.BlockSpec((_TCBLK, D), lambda i: (i, 0)),
            pl.BlockSpec((_TCBLK, D), lambda i: (i, 0)),
        ],
        out_shape=[
            jax.ShapeDtypeStruct((NPAD, D), jnp.float32),
            jax.ShapeDtypeStruct((NPAD, D), jnp.float32),
        ],
    )(acc, z, h, dinv, dinv_next, w, a)


def _conv_final_body(acc_ref, z_ref, h_ref, skip_ref, dinv_ref, w_ref,
                     a_ref, out_ref):
    u = acc_ref[0] + acc_ref[1] + z_ref[...]
    t = dinv_ref[...] * u
    nh = jax.nn.relu(lax.dot_general(
        t, w_ref[...], (((1,), (0,)), ((), ())),
        precision=lax.Precision.HIGHEST, preferred_element_type=jnp.float32))
    a = a_ref[0, 0]
    out_ref[...] = a * nh + (1.0 - a) * h_ref[...] + skip_ref[...]


def _conv_final(acc, z, h, skip, dinv, w, a):
    return pl.pallas_call(
        _conv_final_body,
        grid=(NPAD // _TCBLK,),
        in_specs=[
            pl.BlockSpec((NC, _TCBLK, D), lambda i: (0, i, 0)),
            pl.BlockSpec((_TCBLK, D), lambda i: (i, 0)),
            pl.BlockSpec((_TCBLK, D), lambda i: (i, 0)),
            pl.BlockSpec((_TCBLK, D), lambda i: (i, 0)),
            pl.BlockSpec((_TCBLK, 1), lambda i: (i, 0)),
            pl.BlockSpec((D, D), lambda i: (0, 0)),
            pl.BlockSpec(memory_space=pltpu.SMEM),
        ],
        out_specs=pl.BlockSpec((_TCBLK, D), lambda i: (i, 0)),
        out_shape=jax.ShapeDtypeStruct((NPAD, D), jnp.float32),
    )(acc, z, h, skip, dinv, w, a)


def _pad_idx(idx, k):
    total = NT * k * CHUNK
    idx = jnp.concatenate(
        [idx, jnp.full((total - idx.shape[0],), DUMMY, jnp.int32)])
    return idx.reshape(NT * k, CHUNK)


def kernel(x, edge_index, Wc0, Wc1, Wd0, Wd1, alphas):
    src = edge_index[0].astype(jnp.int32)
    dst = edge_index[1].astype(jnp.int32)

    x_pad = jnp.zeros((NPAD, D), jnp.float32).at[:N].set(x)
    srcf = _pad_idx(src, KF)
    dstf = _pad_idx(dst, KF)
    srch = _pad_idx(src[0::2], KH)
    dsth = _pad_idx(dst[0::2], KH)
    dste = dsth
    dsto = _pad_idx(dst[1::2], KH)

    prop_full = _make_propagate(KF)
    prop_half = _make_propagate(KH)

    cnte, cnto = _make_degree()(dste, dsto)
    dinvf, dinvh, z = _dinv_z(cnte, cnto, x_pad)

    a = jax.nn.sigmoid(alphas.astype(jnp.float32))
    a0 = a[0].reshape(1, 1)
    a1 = a[1].reshape(1, 1)

    acc = prop_full(z, srcf, dstf)
    h1, z = _conv(acc, z, dinvf, dinvf, Wc0)

    acc = prop_full(z, srcf, dstf)
    h2, z = _conv(acc, z, dinvf, dinvf, Wc1)

    acc = prop_full(z, srcf, dstf)
    h3, z = _conv_blend(acc, z, h2, dinvf, dinvh, Wd0, a0)

    acc = prop_half(z, srch, dsth)
    out = _conv_final(acc, z, h3, h2, dinvh, Wd1, a1)

    return out[:N]


# speedup vs baseline: 7.4537x; 1.1937x over previous
import functools

import jax
import jax.numpy as jnp
from jax import lax
from jax.experimental import pallas as pl
from jax.experimental.pallas import tpu as pltpu
from jax.experimental.pallas import tpu_sc as plsc

N = 10000
E = 320000
D = 128
NC, NS = 2, 16
NT = NC * NS
CHUNK = 128
NPAD = 10240
RPT = NPAD // NS
DUMMY = N
KF = 80
KH = 40
DW = 16

_mesh = functools.partial(
    plsc.VectorSubcoreMesh, core_axis_name="c", subcore_axis_name="s",
    num_cores=NC, num_subcores=NS)


def _unpack_idx(packed_v, srow_v, drow_v, j, b):
    mask = jnp.full((16,), 0xFFFF, jnp.int32)
    sh = jnp.full((16,), 16, jnp.int32)
    for i in range(CHUNK // 16):
        v = packed_v[j, pl.ds(i * 16, 16)]
        srow_v[b, pl.ds(i * 16, 16)] = jnp.bitwise_and(v, mask)
        drow_v[b, pl.ds(i * 16, 16)] = lax.shift_right_logical(v, sh)


def _propagate_body(z_hbm, packed_hbm, out_hbm, packed_v, srow_v, drow_v,
                    buf_v, acc_sh, gsem0, gsem1, ssem0, ssem1, K):
    c = lax.axis_index("c")
    s = lax.axis_index("s")
    wid = c * NS + s
    base = s * RPT
    gsems = (gsem0, gsem1)
    ssems = (ssem0, ssem1)

    pltpu.sync_copy(packed_hbm.at[pl.ds(wid * K, K)], packed_v)

    zero16 = jnp.zeros((16,), jnp.float32)

    def _zrow(i, carry):
        for l in range(D // 16):
            buf_v[0, i, pl.ds(l * 16, 16)] = zero16
        return carry

    lax.fori_loop(0, CHUNK, _zrow, 0)
    for b in range(RPT // CHUNK):
        pltpu.sync_copy(buf_v.at[0], acc_sh.at[pl.ds(base + b * CHUNK, CHUNK)])
    plsc.subcore_barrier()

    def _gather(j, b):
        _unpack_idx(packed_v, srow_v, drow_v, j, b)
        pltpu.async_copy(z_hbm.at[srow_v.at[b]], buf_v.at[b], gsems[b])

    def _gwait(b):
        pltpu.make_async_copy(
            z_hbm.at[srow_v.at[b]], buf_v.at[b], gsems[b]).wait()

    def _scatter(b):
        pltpu.async_copy(buf_v.at[b], acc_sh.at[drow_v.at[b]], ssems[b],
                         add=True)

    def _swait(b):
        pltpu.make_async_copy(
            buf_v.at[b], acc_sh.at[drow_v.at[b]], ssems[b]).wait()

    _gather(0, 0)
    _gather(1, 1)
    _gwait(0)
    _scatter(0)

    def _pair(t, carry):
        for b in range(2):
            j = 2 * t + 2 + b
            _swait(b)
            _gather(j, b)
            _gwait(1 - b)
            _scatter(1 - b)
        return carry

    lax.fori_loop(0, (K - 2) // 2, _pair, 0)
    _gwait(1)
    _scatter(1)
    _swait(0)
    _swait(1)
    plsc.subcore_barrier()

    pltpu.sync_copy(acc_sh.at[pl.ds(base, RPT)],
                    out_hbm.at[c, pl.ds(base, RPT)])


def _make_propagate(K):
    body = functools.partial(_propagate_body, K=K)
    return pl.kernel(
        body,
        out_type=jax.ShapeDtypeStruct((NC, NPAD, D), jnp.float32),
        mesh=_mesh(),
        scratch_types=[
            pltpu.VMEM((K, CHUNK), jnp.int32),
            pltpu.VMEM((2, CHUNK), jnp.int32),
            pltpu.VMEM((2, CHUNK), jnp.int32),
            pltpu.VMEM((2, CHUNK, D), jnp.float32),
            pltpu.VMEM_SHARED((NPAD, D), jnp.float32),
            pltpu.SemaphoreType.DMA,
            pltpu.SemaphoreType.DMA,
            pltpu.SemaphoreType.DMA,
            pltpu.SemaphoreType.DMA,
        ],
    )


def _degree_body(dste_hbm, dsto_hbm, out_hbm, dste_v, dsto_v,
                 bufe_v, bufo_v, acc_sh, seme, semo):
    c = lax.axis_index("c")
    s = lax.axis_index("s")
    wid = c * NS + s
    base = s * RPT
    K = KH

    pltpu.sync_copy(dste_hbm.at[pl.ds(wid * K, K)], dste_v)
    pltpu.sync_copy(dsto_hbm.at[pl.ds(wid * K, K)], dsto_v)

    zero16 = jnp.zeros((16,), jnp.float32)
    one16 = jnp.ones((16,), jnp.float32)

    def _zrow(i, carry):
        for l in range(D // 16):
            bufe_v[i, pl.ds(l * 16, 16)] = zero16
        return carry

    lax.fori_loop(0, CHUNK, _zrow, 0)
    for b in range(RPT // CHUNK):
        pltpu.sync_copy(bufe_v, acc_sh.at[pl.ds(base + b * CHUNK, CHUNK)])

    def _prow(i, carry):
        for l in range(D // 16):
            bufe_v[i, pl.ds(l * 16, 16)] = one16 if l < 4 else zero16
            bufo_v[i, pl.ds(l * 16, 16)] = zero16 if l < 4 else one16
        return carry

    lax.fori_loop(0, CHUNK, _prow, 0)
    plsc.subcore_barrier()

    pltpu.async_copy(bufe_v, acc_sh.at[dste_v.at[0]], seme, add=True)
    pltpu.async_copy(bufo_v, acc_sh.at[dsto_v.at[0]], semo, add=True)

    def _edge(j, carry):
        pltpu.make_async_copy(
            bufe_v, acc_sh.at[dste_v.at[j - 1]], seme).wait()
        pltpu.async_copy(bufe_v, acc_sh.at[dste_v.at[j]], seme, add=True)
        pltpu.make_async_copy(
            bufo_v, acc_sh.at[dsto_v.at[j - 1]], semo).wait()
        pltpu.async_copy(bufo_v, acc_sh.at[dsto_v.at[j]], semo, add=True)
        return carry

    lax.fori_loop(1, K, _edge, 0)
    pltpu.make_async_copy(bufe_v, acc_sh.at[dste_v.at[K - 1]], seme).wait()
    pltpu.make_async_copy(bufo_v, acc_sh.at[dsto_v.at[K - 1]], semo).wait()
    plsc.subcore_barrier()

    pltpu.sync_copy(acc_sh.at[pl.ds(base, RPT)],
                    out_hbm.at[c, pl.ds(base, RPT)])


def _make_degree():
    return pl.kernel(
        _degree_body,
        out_type=jax.ShapeDtypeStruct((NC, NPAD, D), jnp.float32),
        mesh=_mesh(),
        scratch_types=[
            pltpu.VMEM((KH, CHUNK), jnp.int32),
            pltpu.VMEM((KH, CHUNK), jnp.int32),
            pltpu.VMEM((CHUNK, D), jnp.float32),
            pltpu.VMEM((CHUNK, D), jnp.float32),
            pltpu.VMEM_SHARED((NPAD, D), jnp.float32),
            pltpu.SemaphoreType.DMA,
            pltpu.SemaphoreType.DMA,
        ],
    )


_TCBLK = 2560


def _dinv_z_body(cnt_ref, x_ref, dinvf_ref, dinvh_ref, z_ref):
    ce = cnt_ref[0, :, 0:1] + cnt_ref[1, :, 0:1]
    co = cnt_ref[0, :, 64:65] + cnt_ref[1, :, 64:65]
    dinvf = lax.rsqrt(ce + co + 1.0)
    dinvh = lax.rsqrt(ce + 1.0)
    dinvf_ref[...] = dinvf
    dinvh_ref[...] = dinvh
    z_ref[...] = dinvf * x_ref[...]


def _dinv_z(cnt, x_pad):
    return pl.pallas_call(
        _dinv_z_body,
        grid=(NPAD // _TCBLK,),
        in_specs=[
            pl.BlockSpec((NC, _TCBLK, D), lambda i: (0, i, 0)),
            pl.BlockSpec((_TCBLK, D), lambda i: (i, 0)),
        ],
        out_specs=[
            pl.BlockSpec((_TCBLK, 1), lambda i: (i, 0)),
            pl.BlockSpec((_TCBLK, 1), lambda i: (i, 0)),
            pl.BlockSpec((_TCBLK, D), lambda i: (i, 0)),
        ],
        out_shape=[
            jax.ShapeDtypeStruct((NPAD, 1), jnp.float32),
            jax.ShapeDtypeStruct((NPAD, 1), jnp.float32),
            jax.ShapeDtypeStruct((NPAD, D), jnp.float32),
        ],
    )(cnt, x_pad)


def _conv_body(acc_ref, z_ref, dinv_ref, dinvn_ref, w_ref, h_ref, zn_ref):
    u = acc_ref[0] + acc_ref[1] + z_ref[...]
    t = dinv_ref[...] * u
    h = jax.nn.relu(lax.dot_general(
        t, w_ref[...], (((1,), (0,)), ((), ())),
        precision=lax.Precision.HIGHEST, preferred_element_type=jnp.float32))
    h_ref[...] = h
    zn_ref[...] = dinvn_ref[...] * h


def _conv(acc, z, dinv, dinv_next, w):
    return pl.pallas_call(
        _conv_body,
        grid=(NPAD // _TCBLK,),
        in_specs=[
            pl.BlockSpec((NC, _TCBLK, D), lambda i: (0, i, 0)),
            pl.BlockSpec((_TCBLK, D), lambda i: (i, 0)),
            pl.BlockSpec((_TCBLK, 1), lambda i: (i, 0)),
            pl.BlockSpec((_TCBLK, 1), lambda i: (i, 0)),
            pl.BlockSpec((D, D), lambda i: (0, 0)),
        ],
        out_specs=[
            pl.BlockSpec((_TCBLK, D), lambda i: (i, 0)),
            pl.BlockSpec((_TCBLK, D), lambda i: (i, 0)),
        ],
        out_shape=[
            jax.ShapeDtypeStruct((NPAD, D), jnp.float32),
            jax.ShapeDtypeStruct((NPAD, D), jnp.float32),
        ],
    )(acc, z, dinv, dinv_next, w)


def _conv_blend_body(acc_ref, z_ref, h_ref, dinv_ref, dinvn_ref, w_ref,
                     a_ref, hn_ref, zn_ref):
    u = acc_ref[0] + acc_ref[1] + z_ref[...]
    t = dinv_ref[...] * u
    nh = jax.nn.relu(lax.dot_general(
        t, w_ref[...], (((1,), (0,)), ((), ())),
        precision=lax.Precision.HIGHEST, preferred_element_type=jnp.float32))
    a = a_ref[0, 0]
    hn = a * nh + (1.0 - a) * h_ref[...]
    hn_ref[...] = hn
    zn_ref[...] = dinvn_ref[...] * hn


def _conv_blend(acc, z, h, dinv, dinv_next, w, a):
    return pl.pallas_call(
        _conv_blend_body,
        grid=(NPAD // _TCBLK,),
        in_specs=[
            pl.BlockSpec((NC, _TCBLK, D), lambda i: (0, i, 0)),
            pl.BlockSpec((_TCBLK, D), lambda i: (i, 0)),
            pl.BlockSpec((_TCBLK, D), lambda i: (i, 0)),
            pl.BlockSpec((_TCBLK, 1), lambda i: (i, 0)),
            pl.BlockSpec((_TCBLK, 1), lambda i: (i, 0)),
            pl.BlockSpec((D, D), lambda i: (0, 0)),
            pl.BlockSpec(memory_space=pltpu.SMEM),
        ],
        out_specs=[
            pl.BlockSpec((_TCBLK, D), lambda i: (i, 0)),
            pl.BlockSpec((_TCBLK, D), lambda i: (i, 0)),
        ],
        out_shape=[
            jax.ShapeDtypeStruct((NPAD, D), jnp.float32),
            jax.ShapeDtypeStruct((NPAD, D), jnp.float32),
        ],
    )(acc, z, h, dinv, dinv_next, w, a)


def _conv_final_body(acc_ref, z_ref, h_ref, skip_ref, dinv_ref, w_ref,
                     a_ref, out_ref):
    u = acc_ref[0] + acc_ref[1] + z_ref[...]
    t = dinv_ref[...] * u
    nh = jax.nn.relu(lax.dot_general(
        t, w_ref[...], (((1,), (0,)), ((), ())),
        precision=lax.Precision.HIGHEST, preferred_element_type=jnp.float32))
    a = a_ref[0, 0]
    out_ref[...] = a * nh + (1.0 - a) * h_ref[...] + skip_ref[...]


def _conv_final(acc, z, h, skip, dinv, w, a):
    return pl.pallas_call(
        _conv_final_body,
        grid=(NPAD // _TCBLK,),
        in_specs=[
            pl.BlockSpec((NC, _TCBLK, D), lambda i: (0, i, 0)),
            pl.BlockSpec((_TCBLK, D), lambda i: (i, 0)),
            pl.BlockSpec((_TCBLK, D), lambda i: (i, 0)),
            pl.BlockSpec((_TCBLK, D), lambda i: (i, 0)),
            pl.BlockSpec((_TCBLK, 1), lambda i: (i, 0)),
            pl.BlockSpec((D, D), lambda i: (0, 0)),
            pl.BlockSpec(memory_space=pltpu.SMEM),
        ],
        out_specs=pl.BlockSpec((_TCBLK, D), lambda i: (i, 0)),
        out_shape=jax.ShapeDtypeStruct((NPAD, D), jnp.float32),
    )(acc, z, h, skip, dinv, w, a)


def _pad_idx(idx, k):
    total = NT * k * CHUNK
    idx = jnp.concatenate(
        [idx, jnp.full((total - idx.shape[0],), DUMMY, jnp.int32)])
    return idx.reshape(NT * k, CHUNK)


def kernel(x, edge_index, Wc0, Wc1, Wd0, Wd1, alphas):
    src = edge_index[0].astype(jnp.int32)
    dst = edge_index[1].astype(jnp.int32)

    x_pad = jnp.zeros((NPAD, D), jnp.float32).at[:N].set(x)
    packf = _pad_idx(src, KF) | (_pad_idx(dst, KF) << 16)
    packh = _pad_idx(src[0::2], KH) | (_pad_idx(dst[0::2], KH) << 16)
    dste = _pad_idx(dst[0::2], KH)
    dsto = _pad_idx(dst[1::2], KH)

    prop_full = _make_propagate(KF)
    prop_half = _make_propagate(KH)

    cnt = _make_degree()(dste, dsto)
    dinvf, dinvh, z = _dinv_z(cnt, x_pad)

    a = jax.nn.sigmoid(alphas.astype(jnp.float32))
    a0 = a[0].reshape(1, 1)
    a1 = a[1].reshape(1, 1)

    acc = prop_full(z, packf)
    h1, z = _conv(acc, z, dinvf, dinvf, Wc0)

    acc = prop_full(z, packf)
    h2, z = _conv(acc, z, dinvf, dinvf, Wc1)

    acc = prop_full(z, packf)
    h3, z = _conv_blend(acc, z, h2, dinvf, dinvh, Wd0, a0)

    acc = prop_half(z, packh)
    out = _conv_final(acc, z, h3, h2, dinvh, Wd1, a1)

    return out[:N]


# speedup vs baseline: 8.4758x; 1.1371x over previous
import functools

import jax
import jax.numpy as jnp
from jax import lax
from jax.experimental import pallas as pl
from jax.experimental.pallas import tpu as pltpu
from jax.experimental.pallas import tpu_sc as plsc

N = 10000
E = 320000
D = 128
NC, NS = 2, 16
NT = NC * NS
CHUNK = 128
NPAD = 10112
RPT = NPAD // NS
DUMMY = N
ROWSF = 2560
ROWSH = 1280
KDEG = ROWSF // NT

_mesh = functools.partial(
    plsc.VectorSubcoreMesh, core_axis_name="c", subcore_axis_name="s",
    num_cores=NC, num_subcores=NS)


def _unpack_idx(packed_v, srow_v, drow_v, j, b):
    mask = jnp.full((16,), 0xFFFF, jnp.int32)
    sh = jnp.full((16,), 16, jnp.int32)
    for i in range(CHUNK // 16):
        v = packed_v[j, pl.ds(i * 16, 16)]
        srow_v[b, pl.ds(i * 16, 16)] = jnp.bitwise_and(v, mask)
        drow_v[b, pl.ds(i * 16, 16)] = lax.shift_right_logical(v, sh)


def _propagate_body(z_hbm, packed_hbm, out_hbm, packed_v, srow_v, drow_v,
                    buf_v, acc_sh, gsem0, gsem1, ssem0, ssem1, KA, KB):
    c = lax.axis_index("c")
    s = lax.axis_index("s")
    base = s * RPT
    gsems = (gsem0, gsem1)
    ssems = (ssem0, ssem1)

    zero16 = jnp.zeros((16,), jnp.float32)

    def _zrow(i, carry):
        for l in range(D // 16):
            buf_v[0, i, pl.ds(l * 16, 16)] = zero16
        return carry

    lax.fori_loop(0, CHUNK, _zrow, 0)
    for b in range(RPT // CHUNK):
        pltpu.sync_copy(buf_v.at[0], acc_sh.at[pl.ds(base + b * CHUNK, CHUNK)])
    _rem = RPT % CHUNK
    if _rem:
        pltpu.sync_copy(
            buf_v.at[0, pl.ds(0, _rem)],
            acc_sh.at[pl.ds(base + (RPT // CHUNK) * CHUNK, _rem)])
    plsc.subcore_barrier()

    def _gather(j, b):
        _unpack_idx(packed_v, srow_v, drow_v, j, b)
        pltpu.async_copy(z_hbm.at[srow_v.at[b]], buf_v.at[b], gsems[b])

    def _gwait(b):
        pltpu.make_async_copy(
            z_hbm.at[srow_v.at[b]], buf_v.at[b], gsems[b]).wait()

    def _scatter(b):
        pltpu.async_copy(buf_v.at[b], acc_sh.at[drow_v.at[b]], ssems[b],
                         add=True)

    def _swait(b):
        pltpu.make_async_copy(
            buf_v.at[b], acc_sh.at[drow_v.at[b]], ssems[b]).wait()

    def _pipeline(K, rowbase):
        pltpu.sync_copy(packed_hbm.at[pl.ds(rowbase, K)],
                        packed_v.at[pl.ds(0, K)])
        _gather(0, 0)
        _gather(1, 1)
        _gwait(0)
        _scatter(0)

        def _pair(t, carry):
            for b in range(2):
                j = 2 * t + 2 + b
                _swait(b)
                _gather(j, b)
                _gwait(1 - b)
                _scatter(1 - b)
            return carry

        lax.fori_loop(0, (K - 2) // 2, _pair, 0)
        _gwait(1)
        _scatter(1)
        _swait(0)
        _swait(1)

    @pl.when(c == 0)
    def _core0():
        _pipeline(KA, s * KA)

    @pl.when(c != 0)
    def _core1():
        _pipeline(KB, NS * KA + s * KB)

    plsc.subcore_barrier()

    pltpu.sync_copy(acc_sh.at[pl.ds(base, RPT)],
                    out_hbm.at[c, pl.ds(base, RPT)])


def _make_propagate(KA, KB):
    body = functools.partial(_propagate_body, KA=KA, KB=KB)
    return pl.kernel(
        body,
        out_type=jax.ShapeDtypeStruct((NC, NPAD, D), jnp.float32),
        mesh=_mesh(),
        scratch_types=[
            pltpu.VMEM((KA, CHUNK), jnp.int32),
            pltpu.VMEM((2, CHUNK), jnp.int32),
            pltpu.VMEM((2, CHUNK), jnp.int32),
            pltpu.VMEM((2, CHUNK, D), jnp.float32),
            pltpu.VMEM_SHARED((NPAD, D), jnp.float32),
            pltpu.SemaphoreType.DMA,
            pltpu.SemaphoreType.DMA,
            pltpu.SemaphoreType.DMA,
            pltpu.SemaphoreType.DMA,
        ],
    )


def _unpack_dst(packed_v, drow_v, j, b):
    sh = jnp.full((16,), 16, jnp.int32)
    for i in range(CHUNK // 16):
        v = packed_v[j, pl.ds(i * 16, 16)]
        drow_v[b, pl.ds(i * 16, 16)] = lax.shift_right_logical(v, sh)


def _degree_body(packed_hbm, out_hbm, packed_v, drow_v, pat_v, acc_sh,
                 sem0, sem1):
    c = lax.axis_index("c")
    s = lax.axis_index("s")
    wid = c * NS + s
    base = s * RPT
    K = KDEG
    sems = (sem0, sem1)

    pltpu.sync_copy(packed_hbm.at[pl.ds(wid * K, K)], packed_v)

    zero16 = jnp.zeros((16,), jnp.float32)
    one16 = jnp.ones((16,), jnp.float32)

    def _zrow(i, carry):
        for l in range(D // 16):
            pat_v[i, pl.ds(l * 16, 16)] = zero16
        return carry

    lax.fori_loop(0, CHUNK, _zrow, 0)
    for b in range(RPT // CHUNK):
        pltpu.sync_copy(pat_v, acc_sh.at[pl.ds(base + b * CHUNK, CHUNK)])
    _rem = RPT % CHUNK
    if _rem:
        pltpu.sync_copy(
            pat_v.at[pl.ds(0, _rem)],
            acc_sh.at[pl.ds(base + (RPT // CHUNK) * CHUNK, _rem)])

    def _prow(t, carry):
        for l in range(D // 16):
            pat_v[2 * t, pl.ds(l * 16, 16)] = one16 if l < 4 else zero16
            pat_v[2 * t + 1, pl.ds(l * 16, 16)] = zero16 if l < 4 else one16
        return carry

    lax.fori_loop(0, CHUNK // 2, _prow, 0)
    plsc.subcore_barrier()

    def _scatter(j, b):
        _unpack_dst(packed_v, drow_v, j, b)
        pltpu.async_copy(pat_v, acc_sh.at[drow_v.at[b]], sems[b], add=True)

    def _swait(b):
        pltpu.make_async_copy(pat_v, acc_sh.at[drow_v.at[b]], sems[b]).wait()

    _scatter(0, 0)
    _scatter(1, 1)

    def _pair(t, carry):
        for b in range(2):
            j = 2 * t + 2 + b
            _swait(b)
            _scatter(j, b)
        return carry

    lax.fori_loop(0, (K - 2) // 2, _pair, 0)
    _swait(0)
    _swait(1)
    plsc.subcore_barrier()

    pltpu.sync_copy(acc_sh.at[pl.ds(base, RPT)],
                    out_hbm.at[c, pl.ds(base, RPT)])


def _make_degree():
    return pl.kernel(
        _degree_body,
        out_type=jax.ShapeDtypeStruct((NC, NPAD, D), jnp.float32),
        mesh=_mesh(),
        scratch_types=[
            pltpu.VMEM((KDEG, CHUNK), jnp.int32),
            pltpu.VMEM((2, CHUNK), jnp.int32),
            pltpu.VMEM((CHUNK, D), jnp.float32),
            pltpu.VMEM_SHARED((NPAD, D), jnp.float32),
            pltpu.SemaphoreType.DMA,
            pltpu.SemaphoreType.DMA,
        ],
    )


_TCBLK = 2528


def _dinv_z_body(cnt_ref, x_ref, dinvf_ref, dinvh_ref, z_ref):
    ce = cnt_ref[0, :, 0:1] + cnt_ref[1, :, 0:1]
    co = cnt_ref[0, :, 64:65] + cnt_ref[1, :, 64:65]
    dinvf = lax.rsqrt(ce + co + 1.0)
    dinvh = lax.rsqrt(ce + 1.0)
    dinvf_ref[...] = dinvf
    dinvh_ref[...] = dinvh
    z_ref[...] = dinvf * x_ref[...]


def _dinv_z(cnt, x_pad):
    return pl.pallas_call(
        _dinv_z_body,
        grid=(NPAD // _TCBLK,),
        in_specs=[
            pl.BlockSpec((NC, _TCBLK, D), lambda i: (0, i, 0)),
            pl.BlockSpec((_TCBLK, D), lambda i: (i, 0)),
        ],
        out_specs=[
            pl.BlockSpec((_TCBLK, 1), lambda i: (i, 0)),
            pl.BlockSpec((_TCBLK, 1), lambda i: (i, 0)),
            pl.BlockSpec((_TCBLK, D), lambda i: (i, 0)),
        ],
        out_shape=[
            jax.ShapeDtypeStruct((NPAD, 1), jnp.float32),
            jax.ShapeDtypeStruct((NPAD, 1), jnp.float32),
            jax.ShapeDtypeStruct((NPAD, D), jnp.float32),
        ],
    )(cnt, x_pad)


def _conv_body(acc_ref, z_ref, dinv_ref, dinvn_ref, w_ref, h_ref, zn_ref):
    u = acc_ref[0] + acc_ref[1] + z_ref[...]
    t = dinv_ref[...] * u
    h = jax.nn.relu(lax.dot_general(
        t, w_ref[...], (((1,), (0,)), ((), ())),
        precision=lax.Precision.HIGHEST, preferred_element_type=jnp.float32))
    h_ref[...] = h
    zn_ref[...] = dinvn_ref[...] * h


def _conv(acc, z, dinv, dinv_next, w):
    return pl.pallas_call(
        _conv_body,
        grid=(NPAD // _TCBLK,),
        in_specs=[
            pl.BlockSpec((NC, _TCBLK, D), lambda i: (0, i, 0)),
            pl.BlockSpec((_TCBLK, D), lambda i: (i, 0)),
            pl.BlockSpec((_TCBLK, 1), lambda i: (i, 0)),
            pl.BlockSpec((_TCBLK, 1), lambda i: (i, 0)),
            pl.BlockSpec((D, D), lambda i: (0, 0)),
        ],
        out_specs=[
            pl.BlockSpec((_TCBLK, D), lambda i: (i, 0)),
            pl.BlockSpec((_TCBLK, D), lambda i: (i, 0)),
        ],
        out_shape=[
            jax.ShapeDtypeStruct((NPAD, D), jnp.float32),
            jax.ShapeDtypeStruct((NPAD, D), jnp.float32),
        ],
    )(acc, z, dinv, dinv_next, w)


def _conv_blend_body(acc_ref, z_ref, h_ref, dinv_ref, dinvn_ref, w_ref,
                     a_ref, hn_ref, zn_ref):
    u = acc_ref[0] + acc_ref[1] + z_ref[...]
    t = dinv_ref[...] * u
    nh = jax.nn.relu(lax.dot_general(
        t, w_ref[...], (((1,), (0,)), ((), ())),
        precision=lax.Precision.HIGHEST, preferred_element_type=jnp.float32))
    a = a_ref[0, 0]
    hn = a * nh + (1.0 - a) * h_ref[...]
    hn_ref[...] = hn
    zn_ref[...] = dinvn_ref[...] * hn


def _conv_blend(acc, z, h, dinv, dinv_next, w, a):
    return pl.pallas_call(
        _conv_blend_body,
        grid=(NPAD // _TCBLK,),
        in_specs=[
            pl.BlockSpec((NC, _TCBLK, D), lambda i: (0, i, 0)),
            pl.BlockSpec((_TCBLK, D), lambda i: (i, 0)),
            pl.BlockSpec((_TCBLK, D), lambda i: (i, 0)),
            pl.BlockSpec((_TCBLK, 1), lambda i: (i, 0)),
            pl.BlockSpec((_TCBLK, 1), lambda i: (i, 0)),
            pl.BlockSpec((D, D), lambda i: (0, 0)),
            pl.BlockSpec(memory_space=pltpu.SMEM),
        ],
        out_specs=[
            pl.BlockSpec((_TCBLK, D), lambda i: (i, 0)),
            pl.BlockSpec((_TCBLK, D), lambda i: (i, 0)),
        ],
        out_shape=[
            jax.ShapeDtypeStruct((NPAD, D), jnp.float32),
            jax.ShapeDtypeStruct((NPAD, D), jnp.float32),
        ],
    )(acc, z, h, dinv, dinv_next, w, a)


def _conv_final_body(acc_ref, z_ref, h_ref, skip_ref, dinv_ref, w_ref,
                     a_ref, out_ref):
    u = acc_ref[0] + acc_ref[1] + z_ref[...]
    t = dinv_ref[...] * u
    nh = jax.nn.relu(lax.dot_general(
        t, w_ref[...], (((1,), (0,)), ((), ())),
        precision=lax.Precision.HIGHEST, preferred_element_type=jnp.float32))
    a = a_ref[0, 0]
    out_ref[...] = a * nh + (1.0 - a) * h_ref[...] + skip_ref[...]


def _conv_final(acc, z, h, skip, dinv, w, a):
    return pl.pallas_call(
        _conv_final_body,
        grid=(NPAD // _TCBLK,),
        in_specs=[
            pl.BlockSpec((NC, _TCBLK, D), lambda i: (0, i, 0)),
            pl.BlockSpec((_TCBLK, D), lambda i: (i, 0)),
            pl.BlockSpec((_TCBLK, D), lambda i: (i, 0)),
            pl.BlockSpec((_TCBLK, D), lambda i: (i, 0)),
            pl.BlockSpec((_TCBLK, 1), lambda i: (i, 0)),
            pl.BlockSpec((D, D), lambda i: (0, 0)),
            pl.BlockSpec(memory_space=pltpu.SMEM),
        ],
        out_specs=pl.BlockSpec((_TCBLK, D), lambda i: (i, 0)),
        out_shape=jax.ShapeDtypeStruct((NPAD, D), jnp.float32),
    )(acc, z, h, skip, dinv, w, a)


def _pad_pack(p, rows):
    fill = jnp.full((rows * CHUNK - p.shape[0],), DUMMY | (DUMMY << 16),
                    jnp.int32)
    return jnp.concatenate([p, fill]).reshape(rows, CHUNK)


def kernel(x, edge_index, Wc0, Wc1, Wd0, Wd1, alphas):
    src = edge_index[0].astype(jnp.int32)
    dst = edge_index[1].astype(jnp.int32)

    x_pad = jnp.zeros((NPAD, D), jnp.float32).at[:N].set(x)
    pack = src | (dst << 16)
    packf = _pad_pack(pack, ROWSF)
    packh = _pad_pack(pack[0::2], ROWSH)

    prop_full = _make_propagate(128, 32)
    prop_half = _make_propagate(64, 16)

    cnt = _make_degree()(packf)
    dinvf, dinvh, z = _dinv_z(cnt, x_pad)

    a = jax.nn.sigmoid(alphas.astype(jnp.float32))
    a0 = a[0].reshape(1, 1)
    a1 = a[1].reshape(1, 1)

    acc = prop_full(z, packf)
    h1, z = _conv(acc, z, dinvf, dinvf, Wc0)

    acc = prop_full(z, packf)
    h2, z = _conv(acc, z, dinvf, dinvf, Wc1)

    acc = prop_full(z, packf)
    h3, z = _conv_blend(acc, z, h2, dinvf, dinvh, Wd0, a0)

    acc = prop_half(z, packh)
    out = _conv_final(acc, z, h3, h2, dinvh, Wd1, a1)

    return out[:N]
